# baseline (speedup 1.0000x reference)
import functools

import jax
import jax.numpy as jnp
from jax import lax
from jax.experimental import pallas as pl
from jax.experimental.pallas import tpu as pltpu
from jax.experimental.pallas import tpu_sc as plsc

NUM_CORES = 2
NUM_SUBCORES = 16
NW = NUM_CORES * NUM_SUBCORES
CHUNK = 128
K = 2
DPAD = 128
D = 64
L = 16


@functools.lru_cache(maxsize=None)
def _make_gather(hist: int, batch: int):
    iw = batch // NW
    nblk = iw // CHUNK
    nchunk = hist * nblk

    mesh = plsc.VectorSubcoreMesh(
        core_axis_name="c", subcore_axis_name="s",
        num_cores=NUM_CORES, num_subcores=NUM_SUBCORES)

    @functools.partial(
        pl.kernel,
        out_type=jax.ShapeDtypeStruct((hist, D, batch), jnp.float32),
        mesh=mesh,
        scratch_types=[
            pltpu.VMEM((nblk, hist, CHUNK), jnp.int32),
            pltpu.VMEM((2, K, CHUNK, DPAD), jnp.float32),
            pltpu.VMEM((2, K, D, CHUNK), jnp.float32),
            pltpu.SemaphoreType.DMA,
            pltpu.SemaphoreType.DMA,
            pltpu.SemaphoreType.DMA,
            pltpu.SemaphoreType.DMA,
        ],
        compiler_params=pltpu.CompilerParams(needs_layout_passes=False),
    )
    def gather_kernel(idx_hbm, table_hbm, out_hbm, idx_v, rows_v, trans_v,
                      sem_in0, sem_in1, sem_out0, sem_out1):
        wid = lax.axis_index("s") * NUM_CORES + lax.axis_index("c")
        i_base = wid * iw
        for blk in range(nblk):
            pltpu.sync_copy(
                idx_hbm.at[:, pl.ds(i_base + blk * CHUNK, CHUNK)],
                idx_v.at[blk])
        sem_in = (sem_in0, sem_in1)
        sem_out = (sem_out0, sem_out1)

        def hblk(g):
            return g // nblk, lax.rem(g, nblk)

        def fire_gather(grp, base):
            for b in range(K):
                h, blk = hblk(base + b)
                pltpu.make_async_copy(
                    table_hbm.at[idx_v.at[blk, h]],
                    rows_v.at[grp, b], sem_in[grp]).start()

        def drain_gather(grp, base):
            for b in range(K):
                h, blk = hblk(base + b)
                pltpu.make_async_copy(
                    table_hbm.at[idx_v.at[blk, h]],
                    rows_v.at[grp, b], sem_in[grp]).wait()

        def out_dst(g):
            h, blk = hblk(g)
            return out_hbm.at[h, :, pl.ds(i_base + blk * CHUNK, CHUNK)]

        def fire_write(grp, base):
            for b in range(K):
                pltpu.make_async_copy(
                    trans_v.at[grp, b], out_dst(base + b), sem_out[grp]).start()

        def drain_write(grp, base):
            for b in range(K):
                pltpu.make_async_copy(
                    trans_v.at[grp, b], out_dst(base + b), sem_out[grp]).wait()

        riota = [lax.iota(jnp.int32, L) + k * L for k in range(D // L)]

        def transpose(grp):
            for b in range(K):
                src = rows_v.at[grp, b]
                dst = trans_v.at[grp, b]

                @pl.loop(0, CHUNK, step=2)
                def _tok(j):
                    for jo in range(2):
                        cvec = jnp.full((L,), 0, jnp.int32) + (j + jo)
                        for k in range(D // L):
                            vals = src[j + jo, pl.ds(k * L, L)]
                            plsc.store_scatter(dst, [riota[k], cvec], vals)

        def half(grp, base, first=False, last=False):
            oth = 1 - grp
            if not first:
                drain_write(oth, base - K)
            if not last:
                fire_gather(oth, base + K)
            drain_gather(grp, base)
            transpose(grp)
            fire_write(grp, base)

        fire_gather(0, 0)
        half(0, 0, first=True)
        half(1, K)

        @pl.loop(2 * K, nchunk - 2 * K, step=2 * K)
        def _steady(g):
            half(0, g)
            half(1, g + K)

        half(0, nchunk - 2 * K)
        half(1, nchunk - K, last=True)
        drain_write(1, nchunk - K)

    return gather_kernel


def kernel(token_ids, weight):
    batch, hist = token_ids.shape
    d = weight.shape[1]
    idx_t = token_ids.astype(jnp.int32).T
    table = jnp.pad(weight, ((0, 0), (0, DPAD - d)))
    out3 = _make_gather(hist, batch)(idx_t, table)
    return out3.transpose(2, 0, 1)


# speedup vs baseline: 1.2179x; 1.2179x over previous
import functools

import jax
import jax.numpy as jnp
from jax import lax
from jax.experimental import pallas as pl
from jax.experimental.pallas import tpu as pltpu
from jax.experimental.pallas import tpu_sc as plsc

NUM_CORES = 2
NUM_SUBCORES = 16
NW = NUM_CORES * NUM_SUBCORES
CHUNK = 128
K = 2
DPAD = 128
D = 64
L = 16


@functools.lru_cache(maxsize=None)
def _make_gather(hist: int, batch: int):
    iw = batch // NW
    nblk = iw // CHUNK
    nchunk = hist * nblk

    mesh = plsc.VectorSubcoreMesh(
        core_axis_name="c", subcore_axis_name="s",
        num_cores=NUM_CORES, num_subcores=NUM_SUBCORES)

    @functools.partial(
        pl.kernel,
        out_type=jax.ShapeDtypeStruct((hist, D, batch), jnp.float32),
        mesh=mesh,
        scratch_types=[
            pltpu.VMEM((nblk, hist, CHUNK), jnp.int32),
            pltpu.VMEM((2, K, CHUNK, DPAD), jnp.float32),
            pltpu.VMEM((2, K, D, CHUNK), jnp.float32),
            pltpu.SemaphoreType.DMA,
            pltpu.SemaphoreType.DMA,
            pltpu.SemaphoreType.DMA,
            pltpu.SemaphoreType.DMA,
        ],
        compiler_params=pltpu.CompilerParams(needs_layout_passes=False),
    )
    def gather_kernel(idx_hbm, table_hbm, out_hbm, idx_v, rows_v, trans_v,
                      sem_in0, sem_in1, sem_out0, sem_out1):
        wid = lax.axis_index("s") * NUM_CORES + lax.axis_index("c")
        i_base = wid * iw
        for blk in range(nblk):
            pltpu.sync_copy(
                idx_hbm.at[:, pl.ds(i_base + blk * CHUNK, CHUNK)],
                idx_v.at[blk])
        sem_in = (sem_in0, sem_in1)
        sem_out = (sem_out0, sem_out1)

        def hblk(g):
            return g // nblk, lax.rem(g, nblk)

        def fire_gather(grp, base):
            for b in range(K):
                h, blk = hblk(base + b)
                pltpu.make_async_copy(
                    table_hbm.at[idx_v.at[blk, h]],
                    rows_v.at[grp, b], sem_in[grp]).start()

        def drain_gather(grp, base):
            for b in range(K):
                h, blk = hblk(base + b)
                pltpu.make_async_copy(
                    table_hbm.at[idx_v.at[blk, h]],
                    rows_v.at[grp, b], sem_in[grp]).wait()

        def out_dst(g):
            h, blk = hblk(g)
            return out_hbm.at[h, :, pl.ds(i_base + blk * CHUNK, CHUNK)]

        def fire_write(grp, base):
            for b in range(K):
                pltpu.make_async_copy(
                    trans_v.at[grp, b], out_dst(base + b), sem_out[grp]).start()

        def drain_write(grp, base):
            for b in range(K):
                pltpu.make_async_copy(
                    trans_v.at[grp, b], out_dst(base + b), sem_out[grp]).wait()

        riota = [lax.iota(jnp.int32, L) + k * L for k in range(D // L)]

        def transpose(grp):
            for b in range(K):
                src = rows_v.at[grp, b]
                dst = trans_v.at[grp, b]

                @plsc.parallel_loop(0, CHUNK, 1, unroll=4)
                def _tok(j):
                    cvec = jnp.full((L,), 0, jnp.int32) + j
                    for k in range(D // L):
                        vals = src[j, pl.ds(k * L, L)]
                        plsc.store_scatter(dst, [riota[k], cvec], vals)

        def half(grp, base, first=False, last=False):
            oth = 1 - grp
            if not first:
                drain_write(oth, base - K)
            if not last:
                fire_gather(oth, base + K)
            drain_gather(grp, base)
            transpose(grp)
            fire_write(grp, base)

        fire_gather(0, 0)
        half(0, 0, first=True)
        half(1, K)

        @pl.loop(2 * K, nchunk - 2 * K, step=2 * K)
        def _steady(g):
            half(0, g)
            half(1, g + K)

        half(0, nchunk - 2 * K)
        half(1, nchunk - K, last=True)
        drain_write(1, nchunk - K)

    return gather_kernel


def kernel(token_ids, weight):
    batch, hist = token_ids.shape
    d = weight.shape[1]
    idx_t = token_ids.astype(jnp.int32).T
    table = jnp.pad(weight, ((0, 0), (0, DPAD - d)))
    out3 = _make_gather(hist, batch)(idx_t, table)
    return out3.transpose(2, 0, 1)


# speedup vs baseline: 1.2183x; 1.0003x over previous
import functools

import jax
import jax.numpy as jnp
from jax import lax
from jax.experimental import pallas as pl
from jax.experimental.pallas import tpu as pltpu
from jax.experimental.pallas import tpu_sc as plsc

NUM_CORES = 2
NUM_SUBCORES = 16
NW = NUM_CORES * NUM_SUBCORES
CHUNK = 128
K = 2
DPAD = 128
D = 64
L = 16


@functools.lru_cache(maxsize=None)
def _make_gather(hist: int, batch: int):
    iw = batch // NW
    nblk = iw // CHUNK
    nchunk = hist * nblk

    mesh = plsc.VectorSubcoreMesh(
        core_axis_name="c", subcore_axis_name="s",
        num_cores=NUM_CORES, num_subcores=NUM_SUBCORES)

    @functools.partial(
        pl.kernel,
        out_type=jax.ShapeDtypeStruct((hist, D, batch), jnp.float32),
        mesh=mesh,
        scratch_types=[
            pltpu.VMEM((nblk, hist, CHUNK), jnp.int32),
            pltpu.VMEM((2, K, CHUNK, DPAD), jnp.float32),
            pltpu.VMEM((2, K, D, CHUNK), jnp.float32),
            pltpu.SemaphoreType.DMA,
            pltpu.SemaphoreType.DMA,
            pltpu.SemaphoreType.DMA,
            pltpu.SemaphoreType.DMA,
        ],
        compiler_params=pltpu.CompilerParams(needs_layout_passes=False),
    )
    def gather_kernel(idx_hbm, table_hbm, out_hbm, idx_v, rows_v, trans_v,
                      sem_in0, sem_in1, sem_out0, sem_out1):
        wid = lax.axis_index("s") * NUM_CORES + lax.axis_index("c")
        i_base = wid * iw
        for blk in range(nblk):
            pltpu.sync_copy(
                idx_hbm.at[:, pl.ds(i_base + blk * CHUNK, CHUNK)],
                idx_v.at[blk])
        sem_in = (sem_in0, sem_in1)
        sem_out = (sem_out0, sem_out1)

        def hblk(g):
            return g // nblk, lax.rem(g, nblk)

        def fire_gather(grp, base):
            for b in range(K):
                h, blk = hblk(base + b)
                pltpu.make_async_copy(
                    table_hbm.at[idx_v.at[blk, h]],
                    rows_v.at[grp, b], sem_in[grp]).start()

        def drain_gather(grp, base):
            for b in range(K):
                h, blk = hblk(base + b)
                pltpu.make_async_copy(
                    table_hbm.at[idx_v.at[blk, h]],
                    rows_v.at[grp, b], sem_in[grp]).wait()

        def out_dst(g):
            h, blk = hblk(g)
            return out_hbm.at[h, :, pl.ds(i_base + blk * CHUNK, CHUNK)]

        def fire_write(grp, base):
            for b in range(K):
                pltpu.make_async_copy(
                    trans_v.at[grp, b], out_dst(base + b), sem_out[grp]).start()

        def drain_write(grp, base):
            for b in range(K):
                pltpu.make_async_copy(
                    trans_v.at[grp, b], out_dst(base + b), sem_out[grp]).wait()

        riota = [lax.iota(jnp.int32, L) + k * L for k in range(D // L)]

        def transpose(grp):
            for b in range(K):
                src = rows_v.at[grp, b]
                dst = trans_v.at[grp, b]

                @plsc.parallel_loop(0, CHUNK, 1, unroll=8)
                def _tok(j):
                    cvec = jnp.full((L,), 0, jnp.int32) + j
                    for k in range(D // L):
                        vals = src[j, pl.ds(k * L, L)]
                        plsc.store_scatter(dst, [riota[k], cvec], vals)

        def half(grp, base, first=False, last=False):
            oth = 1 - grp
            if not first:
                drain_write(oth, base - K)
            if not last:
                fire_gather(oth, base + K)
            drain_gather(grp, base)
            transpose(grp)
            fire_write(grp, base)

        fire_gather(0, 0)
        half(0, 0, first=True)
        half(1, K)

        @pl.loop(2 * K, nchunk - 2 * K, step=2 * K)
        def _steady(g):
            half(0, g)
            half(1, g + K)

        half(0, nchunk - 2 * K)
        half(1, nchunk - K, last=True)
        drain_write(1, nchunk - K)

    return gather_kernel


def kernel(token_ids, weight):
    batch, hist = token_ids.shape
    d = weight.shape[1]
    idx_t = token_ids.astype(jnp.int32).T
    table = jnp.pad(weight, ((0, 0), (0, DPAD - d)))
    out3 = _make_gather(hist, batch)(idx_t, table)
    return out3.transpose(2, 0, 1)
